# baseline (speedup 1.0000x reference)
import jax
import jax.numpy as jnp
from jax import lax
from jax.experimental import pallas as pl
from jax.experimental.pallas import tpu as pltpu
from jax.experimental.pallas import tpu_sc as plsc

H_DIM = 128
N_HN = 100000
N_HE = 320000

NC = 2
NS = 16
NW = NC * NS

CHUNK = 64
NBUF = 10

N_CHUNKS_N = 50
N_CHUNKS_E = 160
N_PAD = NW * N_CHUNKS_N * CHUNK
E_PAD = NW * N_CHUNKS_E * CHUNK


def _emb_kernel(hn_hbm, he_hbm, n_table_hbm, e_table_hbm,
                n_out_hbm, e_out_hbm,
                idx_n, idx_e, rows, etab_sp, gsems, osems, isem):
    wid = lax.axis_index("s") * NC + lax.axis_index("c")

    @pl.when(lax.axis_index("s") == 0)
    def _():
        pltpu.sync_copy(e_table_hbm, etab_sp)

    nn = N_CHUNKS_N * CHUNK
    ne = N_CHUNKS_E * CHUNK
    pltpu.async_copy(hn_hbm.at[pl.ds(wid * nn, nn)], idx_n, isem)
    pltpu.make_async_copy(hn_hbm.at[pl.ds(0, nn)], idx_n, isem).wait()
    pltpu.async_copy(he_hbm.at[pl.ds(wid * ne, ne)], idx_e, isem)
    pltpu.make_async_copy(he_hbm.at[pl.ds(0, ne)], idx_e, isem).wait()
    plsc.subcore_barrier()

    def run_table(idx_v, table_hbm, out_hbm, n_chunks, n_rows):
        base = wid * (n_chunks * CHUNK)
        n_outer = n_chunks // NBUF

        def start_gather(j, b):
            pltpu.async_copy(table_hbm.at[idx_v.at[pl.ds(j * CHUNK, CHUNK)]],
                             rows.at[b], gsems[b])

        def wait_gather(b):
            pltpu.make_async_copy(table_hbm.at[idx_v.at[pl.ds(0, CHUNK)]],
                                  rows.at[b], gsems[b]).wait()

        def store_off(j):
            return jnp.minimum(base + j * CHUNK, n_rows - CHUNK)

        def start_store(j, b):
            pltpu.async_copy(rows.at[b], out_hbm.at[pl.ds(store_off(j),
                                                          CHUNK)], osems[b])

        def wait_store(b):
            pltpu.make_async_copy(rows.at[b],
                                  out_hbm.at[pl.ds(0, CHUNK)],
                                  osems[b]).wait()

        assert n_chunks % NBUF == 0
        for b in range(NBUF):
            start_gather(b, b)
        wait_gather(0)
        start_store(0, 0)

        def outer(g, _):
            j0 = g * NBUF
            for k in range(NBUF):
                j = j0 + k + 1
                wait_store(k)
                start_gather(j - 1 + NBUF, k)
                wait_gather((k + 1) % NBUF)
                start_store(j, (k + 1) % NBUF)
            return 0

        lax.fori_loop(0, n_outer - 1, outer, 0)

        for j in range(n_chunks - NBUF + 1, n_chunks):
            wait_store((j - 1) % NBUF)
            wait_gather(j % NBUF)
            start_store(j, j % NBUF)
        wait_store((n_chunks - 1) % NBUF)

    run_table(idx_n, n_table_hbm, n_out_hbm, N_CHUNKS_N, N_HN)
    run_table(idx_e, etab_sp, e_out_hbm, N_CHUNKS_E, N_HE)


@jax.jit
def _run(hn_pad, he_pad, n_table, e_table):
    mesh = plsc.VectorSubcoreMesh(core_axis_name="c", subcore_axis_name="s")
    f = pl.kernel(
        _emb_kernel,
        out_type=(
            jax.ShapeDtypeStruct((N_HN, H_DIM), jnp.float32),
            jax.ShapeDtypeStruct((N_HE, H_DIM), jnp.float32),
        ),
        mesh=mesh,
        scratch_types=[
            pltpu.VMEM((N_CHUNKS_N * CHUNK,), jnp.int32),
            pltpu.VMEM((N_CHUNKS_E * CHUNK,), jnp.int32),
            pltpu.VMEM((NBUF, CHUNK, H_DIM), jnp.float32),
            pltpu.VMEM_SHARED((64, H_DIM), jnp.float32),
            [pltpu.SemaphoreType.DMA] * NBUF,
            [pltpu.SemaphoreType.DMA] * NBUF,
            pltpu.SemaphoreType.DMA,
        ],
    )
    return f(hn_pad, he_pad, n_table, e_table)


def _pad_tail_replicated(idx, total, pad_len):
    n_keep = (total // CHUNK) * CHUNK
    if n_keep == total:
        head = idx
    else:
        head = idx[:n_keep]
    tail = jnp.tile(idx[total - CHUNK:total], (pad_len + CHUNK - 1) // CHUNK)
    return jnp.concatenate([head, tail])


def kernel(g, hn, r, he, norm, n_table, e_table):
    hn_flat = hn.reshape(-1).astype(jnp.int32)
    he_flat = he.reshape(-1).astype(jnp.int32)
    hn_pad = _pad_tail_replicated(hn_flat, N_HN, N_PAD - (N_HN // CHUNK) * CHUNK)
    he_pad = _pad_tail_replicated(he_flat, N_HE, E_PAD - N_HE)
    assert hn_pad.shape[0] == N_PAD and he_pad.shape[0] == E_PAD
    return _run(hn_pad, he_pad, n_table, e_table)


# speedup vs baseline: 1.0648x; 1.0648x over previous
import jax
import jax.numpy as jnp
from jax import lax
from jax.experimental import pallas as pl
from jax.experimental.pallas import tpu as pltpu
from jax.experimental.pallas import tpu_sc as plsc

H_DIM = 128
N_HN = 100000
N_HE = 320000

NC = 2
NS = 16
NW = NC * NS

CHUNK = 128
NBUF = 5

N_CHUNKS_N = 25
N_CHUNKS_E = 80
N_PAD = NW * N_CHUNKS_N * CHUNK
E_PAD = NW * N_CHUNKS_E * CHUNK


def _emb_kernel(hn_hbm, he_hbm, n_table_hbm, e_table_hbm,
                n_out_hbm, e_out_hbm,
                idx_n, idx_e, rows, etab_sp, gsems, osems, isem):
    wid = lax.axis_index("s") * NC + lax.axis_index("c")

    @pl.when(lax.axis_index("s") == 0)
    def _():
        pltpu.sync_copy(e_table_hbm, etab_sp)

    nn = N_CHUNKS_N * CHUNK
    ne = N_CHUNKS_E * CHUNK
    pltpu.async_copy(hn_hbm.at[pl.ds(wid * nn, nn)], idx_n, isem)
    pltpu.make_async_copy(hn_hbm.at[pl.ds(0, nn)], idx_n, isem).wait()
    pltpu.async_copy(he_hbm.at[pl.ds(wid * ne, ne)], idx_e, isem)
    pltpu.make_async_copy(he_hbm.at[pl.ds(0, ne)], idx_e, isem).wait()
    plsc.subcore_barrier()

    def run_table(idx_v, table_hbm, out_hbm, n_chunks, n_rows):
        base = wid * (n_chunks * CHUNK)
        n_outer = n_chunks // NBUF

        def start_gather(j, b):
            pltpu.async_copy(table_hbm.at[idx_v.at[pl.ds(j * CHUNK, CHUNK)]],
                             rows.at[b], gsems[b])

        def wait_gather(b):
            pltpu.make_async_copy(table_hbm.at[idx_v.at[pl.ds(0, CHUNK)]],
                                  rows.at[b], gsems[b]).wait()

        def store_off(j):
            return jnp.minimum(base + j * CHUNK, n_rows - CHUNK)

        def start_store(j, b):
            pltpu.async_copy(rows.at[b], out_hbm.at[pl.ds(store_off(j),
                                                          CHUNK)], osems[b])

        def wait_store(b):
            pltpu.make_async_copy(rows.at[b],
                                  out_hbm.at[pl.ds(0, CHUNK)],
                                  osems[b]).wait()

        assert n_chunks % NBUF == 0
        for b in range(NBUF):
            start_gather(b, b)
        wait_gather(0)
        start_store(0, 0)

        def outer(g, _):
            j0 = g * NBUF
            for k in range(NBUF):
                j = j0 + k + 1
                wait_store(k)
                start_gather(j - 1 + NBUF, k)
                wait_gather((k + 1) % NBUF)
                start_store(j, (k + 1) % NBUF)
            return 0

        lax.fori_loop(0, n_outer - 1, outer, 0)

        for j in range(n_chunks - NBUF + 1, n_chunks):
            wait_store((j - 1) % NBUF)
            wait_gather(j % NBUF)
            start_store(j, j % NBUF)
        wait_store((n_chunks - 1) % NBUF)

    run_table(idx_n, n_table_hbm, n_out_hbm, N_CHUNKS_N, N_HN)
    run_table(idx_e, etab_sp, e_out_hbm, N_CHUNKS_E, N_HE)


@jax.jit
def _run(hn_pad, he_pad, n_table, e_table):
    mesh = plsc.VectorSubcoreMesh(core_axis_name="c", subcore_axis_name="s")
    f = pl.kernel(
        _emb_kernel,
        out_type=(
            jax.ShapeDtypeStruct((N_HN, H_DIM), jnp.float32),
            jax.ShapeDtypeStruct((N_HE, H_DIM), jnp.float32),
        ),
        mesh=mesh,
        scratch_types=[
            pltpu.VMEM((N_CHUNKS_N * CHUNK,), jnp.int32),
            pltpu.VMEM((N_CHUNKS_E * CHUNK,), jnp.int32),
            pltpu.VMEM((NBUF, CHUNK, H_DIM), jnp.float32),
            pltpu.VMEM_SHARED((64, H_DIM), jnp.float32),
            [pltpu.SemaphoreType.DMA] * NBUF,
            [pltpu.SemaphoreType.DMA] * NBUF,
            pltpu.SemaphoreType.DMA,
        ],
    )
    return f(hn_pad, he_pad, n_table, e_table)


def _pad_tail_replicated(idx, total, pad_len):
    n_keep = (total // CHUNK) * CHUNK
    if n_keep == total:
        head = idx
    else:
        head = idx[:n_keep]
    tail = jnp.tile(idx[total - CHUNK:total], (pad_len + CHUNK - 1) // CHUNK)
    return jnp.concatenate([head, tail])


def kernel(g, hn, r, he, norm, n_table, e_table):
    hn_flat = hn.reshape(-1).astype(jnp.int32)
    he_flat = he.reshape(-1).astype(jnp.int32)
    hn_pad = _pad_tail_replicated(hn_flat, N_HN, N_PAD - (N_HN // CHUNK) * CHUNK)
    he_pad = _pad_tail_replicated(he_flat, N_HE, E_PAD - N_HE)
    assert hn_pad.shape[0] == N_PAD and he_pad.shape[0] == E_PAD
    return _run(hn_pad, he_pad, n_table, e_table)
